# baseline (device time: 78778 ns/iter reference)
import jax
import jax.numpy as jnp
from jax import lax
from jax.experimental import pallas as pl
from jax.experimental.pallas import tpu as pltpu

N_DEV = 16


def kernel(x, w_mat):
    m, k_per = x.shape
    k, n = w_mat.shape
    m_per = m // N_DEV

    def body(x_ref, w_ref, o_ref, comm_ref, send_sems, recv_sems):
        s = pl.program_id(0)
        me = lax.axis_index("i")

        @pl.when(s == 0)
        def _():
            barrier_sem = pltpu.get_barrier_semaphore()
            for j in range(N_DEV):
                @pl.when(me != j)
                def _():
                    pl.semaphore_signal(
                        barrier_sem, inc=1,
                        device_id=(j,), device_id_type=pl.DeviceIdType.MESH,
                    )
            pl.semaphore_wait(barrier_sem, N_DEV - 1)
            for j in range(N_DEV):
                @pl.when(me != j)
                def _():
                    pltpu.make_async_remote_copy(
                        src_ref=x_ref.at[pl.ds(j * m_per, m_per), :],
                        dst_ref=comm_ref.at[me],
                        send_sem=send_sems.at[j],
                        recv_sem=recv_sems.at[me],
                        device_id=(j,),
                        device_id_type=pl.DeviceIdType.MESH,
                    ).start()

        @pl.when(s != me)
        def _():
            pltpu.make_async_remote_copy(
                src_ref=comm_ref.at[s],
                dst_ref=comm_ref.at[s],
                send_sem=send_sems.at[s],
                recv_sem=recv_sems.at[s],
                device_id=(0,),
                device_id_type=pl.DeviceIdType.MESH,
            ).wait_recv()

        local_blk = x_ref[pl.ds(me * m_per, m_per), :]
        blk = jnp.where(s == me, local_blk, comm_ref[s])
        acc = jnp.dot(blk, w_ref[...], preferred_element_type=jnp.float32)

        @pl.when(s == 0)
        def _():
            o_ref[...] = acc

        @pl.when(s > 0)
        def _():
            o_ref[...] += acc

        @pl.when(s == N_DEV - 1)
        def _():
            for j in range(N_DEV):
                @pl.when(me != j)
                def _():
                    pltpu.make_async_remote_copy(
                        src_ref=x_ref.at[pl.ds(j * m_per, m_per), :],
                        dst_ref=comm_ref.at[me],
                        send_sem=send_sems.at[j],
                        recv_sem=recv_sems.at[me],
                        device_id=(j,),
                        device_id_type=pl.DeviceIdType.MESH,
                    ).wait_send()

    return pl.pallas_call(
        body,
        grid=(N_DEV,),
        out_shape=jax.ShapeDtypeStruct((m_per, n), jnp.float32),
        in_specs=[
            pl.BlockSpec((m, k_per), lambda s: (0, 0)),
            pl.BlockSpec((k // N_DEV, n), lambda s: (s, 0)),
        ],
        out_specs=pl.BlockSpec((m_per, n), lambda s: (0, 0)),
        scratch_shapes=[
            pltpu.VMEM((N_DEV, m_per, k_per), x.dtype),
            pltpu.SemaphoreType.DMA((N_DEV,)),
            pltpu.SemaphoreType.DMA((N_DEV,)),
        ],
        compiler_params=pltpu.CompilerParams(
            dimension_semantics=("arbitrary",),
            collective_id=0,
        ),
    )(x, w_mat)
